# baseline (speedup 1.0000x reference)
import jax
import jax.numpy as jnp
from jax import lax
from jax.experimental import pallas as pl
from jax.experimental.pallas import tpu as pltpu


def _stages(n):
    out = []
    kk = 2
    while kk <= n:
        j = kk // 2
        while j >= 1:
            out.append((kk, j))
            j //= 2
        kk *= 2
    return out


def _partner(a, j):
    n, c = a.shape
    r = a.reshape(n // (2 * j), 2 * j, c)
    p = jnp.concatenate([r[:, j:], r[:, :j]], axis=1)
    return p.reshape(n, c)


def _bitonic(key, idx, payloads):
    n, c = key.shape
    row = lax.broadcasted_iota(jnp.int32, (n, c), 0)
    arrs = [key] + ([idx] if idx is not None else []) + list(payloads)
    for kk, j in _stages(n):
        partners = [_partner(a, j) for a in arrs]
        pk = partners[0]
        if idx is not None:
            pidx = partners[1]
            t = (arrs[0] > pk) | ((arrs[0] == pk) & (arrs[1] > pidx))
        else:
            t = arrs[0] > pk
        is_lo = (row & j) == 0
        asc = (row & kk) == 0
        take = (t == is_lo) == asc
        arrs = [jnp.where(take, p, a) for a, p in zip(arrs, partners)]
    return arrs


def _sort_kernel(q_ref, k_ref, v_ref, o_ref):
    q = q_ref[0]
    k = k_ref[0]
    v = v_ref[0]
    n, c = q.shape
    idx = lax.broadcasted_iota(jnp.int32, (n, c), 0)
    _, _, w = _bitonic(k, idx, [v])
    _, q_idx = _bitonic(q, idx, [])
    _, out = _bitonic(q_idx, None, [w])
    o_ref[0] = out


def _pack(x):
    b, h, n, c = x.shape
    g = b * h // 2
    return x.reshape(g, 2, n, c).transpose(0, 2, 1, 3).reshape(g, n, 2 * c)


def _unpack(y, b, h, c):
    g, n, c2 = y.shape
    return y.reshape(g, n, 2, c).transpose(0, 2, 1, 3).reshape(b, h, n, c)


def kernel(q, k, v):
    b, h, n, c = q.shape
    qp, kp, vp = _pack(q), _pack(k), _pack(v)
    g, _, lanes = qp.shape
    out = pl.pallas_call(
        _sort_kernel,
        grid=(g,),
        in_specs=[pl.BlockSpec((1, n, lanes), lambda i: (i, 0, 0))] * 3,
        out_specs=pl.BlockSpec((1, n, lanes), lambda i: (i, 0, 0)),
        out_shape=jax.ShapeDtypeStruct((g, n, lanes), jnp.float32),
        compiler_params=pltpu.CompilerParams(
            dimension_semantics=("parallel",)),
    )(qp, kp, vp)
    o = _unpack(out, b, h, c)
    return (o, o)


# speedup vs baseline: 1.3541x; 1.3541x over previous
import jax
import jax.numpy as jnp
from jax import lax
from jax.experimental import pallas as pl
from jax.experimental.pallas import tpu as pltpu


def _stages(n):
    out = []
    kk = 2
    while kk <= n:
        j = kk // 2
        while j >= 1:
            out.append((kk, j))
            j //= 2
        kk *= 2
    return out


def _partner(a, j, is_lo):
    n, c = a.shape
    if j >= 8:
        r = a.reshape(n // (2 * j), 2 * j, c)
        return jnp.concatenate([r[:, j:], r[:, :j]], axis=1).reshape(n, c)
    up = jnp.concatenate([a[j:], a[:j]], axis=0)
    dn = jnp.concatenate([a[n - j:], a[:n - j]], axis=0)
    return jnp.where(is_lo, up, dn)


def _bitonic_multi(groups, row):
    n, _ = groups[0][0].shape
    for kk, j in _stages(n):
        is_lo = (row & j) == 0
        flip = is_lo == ((row & kk) == 0)
        new_groups = []
        for key, idx, pays in groups:
            arrs = [key] + ([idx] if idx is not None else []) + list(pays)
            parts = [_partner(a, j, is_lo) for a in arrs]
            pk = parts[0]
            if idx is not None:
                t = (key > pk) | ((key == pk) & (idx > parts[1]))
            else:
                t = key > pk
            take = t == flip
            out = [jnp.where(take, p, a) for a, p in zip(arrs, parts)]
            ni = 1 if idx is not None else 0
            new_groups.append(
                (out[0], out[1] if idx is not None else None, out[1 + ni:]))
        groups = new_groups
    return groups


def _bitonic(key, idx, payloads):
    n, _ = key.shape
    row = lax.broadcasted_iota(jnp.int32, key.shape, 0)
    key, idx, pays = _bitonic_multi([(key, idx, list(payloads))], row)[0]
    return [key] + ([idx] if idx is not None else []) + list(pays)


def _sort_kernel(q_ref, k_ref, v_ref, o_ref):
    q = q_ref[0]
    k = k_ref[0]
    v = v_ref[0]
    n, c = q.shape
    row = lax.broadcasted_iota(jnp.int32, (n, c), 0)
    (_, _, (w,)), = _bitonic_multi([(k, row, [v])], row)
    (_, q_idx, _), = _bitonic_multi([(q, row, [])], row)
    (_, _, (out,)), = _bitonic_multi([(q_idx, None, [w])], row)
    o_ref[0] = out


def _pack(x):
    b, h, n, c = x.shape
    g = b * h // 2
    return x.reshape(g, 2, n, c).transpose(0, 2, 1, 3).reshape(g, n, 2 * c)


def _unpack(y, b, h, c):
    g, n, c2 = y.shape
    return y.reshape(g, n, 2, c).transpose(0, 2, 1, 3).reshape(b, h, n, c)


def kernel(q, k, v):
    b, h, n, c = q.shape
    qp, kp, vp = _pack(q), _pack(k), _pack(v)
    g, _, lanes = qp.shape
    out = pl.pallas_call(
        _sort_kernel,
        grid=(g,),
        in_specs=[pl.BlockSpec((1, n, lanes), lambda i: (i, 0, 0))] * 3,
        out_specs=pl.BlockSpec((1, n, lanes), lambda i: (i, 0, 0)),
        out_shape=jax.ShapeDtypeStruct((g, n, lanes), jnp.float32),
        compiler_params=pltpu.CompilerParams(
            dimension_semantics=("parallel",),
            vmem_limit_bytes=128 * 1024 * 1024),
    )(qp, kp, vp)
    o = _unpack(out, b, h, c)
    return (o, o)


# speedup vs baseline: 1.7336x; 1.2803x over previous
import functools

import jax
import jax.numpy as jnp
from jax import lax
from jax.experimental import pallas as pl
from jax.experimental.pallas import tpu as pltpu
from jax.experimental.pallas import tpu_sc as plsc


def _stages(n):
    out = []
    kk = 2
    while kk <= n:
        j = kk // 2
        while j >= 1:
            out.append((kk, j))
            j //= 2
        kk *= 2
    return out


def _partner(a, j, is_lo):
    n, c = a.shape
    if j >= 8:
        r = a.reshape(n // (2 * j), 2 * j, c)
        return jnp.concatenate([r[:, j:], r[:, :j]], axis=1).reshape(n, c)
    up = jnp.concatenate([a[j:], a[:j]], axis=0)
    dn = jnp.concatenate([a[n - j:], a[:n - j]], axis=0)
    return jnp.where(is_lo, up, dn)


def _bitonic_multi(groups, row):
    n, _ = groups[0][0].shape
    for kk, j in _stages(n):
        is_lo = (row & j) == 0
        flip = is_lo == ((row & kk) == 0)
        new_groups = []
        for key, idx, pays in groups:
            arrs = [key] + ([idx] if idx is not None else []) + list(pays)
            parts = [_partner(a, j, is_lo) for a in arrs]
            pk = parts[0]
            if idx is not None:
                t = (key > pk) | ((key == pk) & (idx > parts[1]))
            else:
                t = key > pk
            take = t == flip
            out = [jnp.where(take, p, a) for a, p in zip(arrs, parts)]
            ni = 1 if idx is not None else 0
            new_groups.append(
                (out[0], out[1] if idx is not None else None, out[1 + ni:]))
        groups = new_groups
    return groups


def _bitonic(key, idx, payloads):
    n, _ = key.shape
    row = lax.broadcasted_iota(jnp.int32, key.shape, 0)
    key, idx, pays = _bitonic_multi([(key, idx, list(payloads))], row)[0]
    return [key] + ([idx] if idx is not None else []) + list(pays)


def _sort_kernel(q_ref, k_ref, v_ref, w_ref, qi_ref):
    q = q_ref[0]
    k = k_ref[0]
    v = v_ref[0]
    n, c = q.shape
    row = lax.broadcasted_iota(jnp.int32, (n, c), 0)
    (_, _, (w,)), = _bitonic_multi([(k, row, [v])], row)
    (_, q_idx, _), = _bitonic_multi([(q, row, [])], row)
    w_ref[0] = w
    qi_ref[0] = q_idx


def _make_sc_permute(rows, n):
    info = plsc.get_sparse_core_info()
    nc, ns, lanes = info.num_cores, info.num_subcores, info.num_lanes
    nw = nc * ns
    rows_per = rows // nw
    mesh = plsc.VectorSubcoreMesh(core_axis_name="c", subcore_axis_name="s")

    @functools.partial(
        pl.kernel, mesh=mesh,
        compiler_params=pltpu.CompilerParams(needs_layout_passes=False),
        out_type=jax.ShapeDtypeStruct((rows, n), jnp.float32),
        scratch_types=[
            pltpu.VMEM((n,), jnp.float32),
            pltpu.VMEM((n,), jnp.int32),
            pltpu.VMEM((n,), jnp.float32),
        ],
    )
    def sc_permute(w_hbm, idx_hbm, out_hbm, w_v, idx_v, o_v):
        wid = lax.axis_index("s") * nc + lax.axis_index("c")

        def body(i, carry):
            r = wid * rows_per + i
            pltpu.sync_copy(w_hbm.at[r], w_v)
            pltpu.sync_copy(idx_hbm.at[r], idx_v)

            def inner(t, carry2):
                ind = idx_v[pl.ds(t * lanes, lanes)]
                x = w_v[pl.ds(t * lanes, lanes)]
                plsc.store_scatter(o_v, [ind], x)
                return carry2

            lax.fori_loop(0, n // lanes, inner, 0)
            pltpu.sync_copy(o_v, out_hbm.at[r])
            return carry

        lax.fori_loop(0, rows_per, body, 0)

    return sc_permute


def _pack(x):
    b, h, n, c = x.shape
    g = b * h // 2
    return x.reshape(g, 2, n, c).transpose(0, 2, 1, 3).reshape(g, n, 2 * c)


def _unpack(y, b, h, c):
    g, n, c2 = y.shape
    return y.reshape(g, n, 2, c).transpose(0, 2, 1, 3).reshape(b, h, n, c)


def kernel(q, k, v):
    b, h, n, c = q.shape
    qp, kp, vp = _pack(q), _pack(k), _pack(v)
    g, _, lanes = qp.shape
    w, q_idx = pl.pallas_call(
        _sort_kernel,
        grid=(g,),
        in_specs=[pl.BlockSpec((1, n, lanes), lambda i: (i, 0, 0))] * 3,
        out_specs=[pl.BlockSpec((1, n, lanes), lambda i: (i, 0, 0))] * 2,
        out_shape=[
            jax.ShapeDtypeStruct((g, n, lanes), jnp.float32),
            jax.ShapeDtypeStruct((g, n, lanes), jnp.int32),
        ],
        compiler_params=pltpu.CompilerParams(
            dimension_semantics=("parallel",),
            vmem_limit_bytes=128 * 1024 * 1024),
    )(qp, kp, vp)
    rows = g * lanes
    w_t = w.transpose(0, 2, 1).reshape(rows, n)
    qi_t = q_idx.transpose(0, 2, 1).reshape(rows, n)
    out_t = _make_sc_permute(rows, n)(w_t, qi_t)
    o = (out_t.reshape(g, 2, c, n).transpose(0, 1, 3, 2)
         .reshape(b, h, n, c))
    return (o, o)
